# baseline (speedup 1.0000x reference)
import functools

import jax
import jax.numpy as jnp
from jax import lax
from jax.experimental import pallas as pl
from jax.experimental.pallas import tpu as pltpu
from jax.experimental.pallas import tpu_sc as plsc

_NUM_CORES = 2
_NUM_SUBCORES = 16
_NW = _NUM_CORES * _NUM_SUBCORES
_L = 16

_VOCAB = 1000000
_DIM = 64
_NBLK = (_VOCAB + 127) // 128
_Q, _R = divmod(_NBLK, _NW)

_CHSZ = 12800
_CAP = 8192
_GB = 3
_GW = _GB * 128
_NGRP = (_Q + 1 + _GB - 1) // _GB


def _emb_kernel(n, table_t, idx_hbm, out_hbm,
                idx_v, sel_i, sel_p, srt_i, srt_p,
                hist_v, offs_v, cur_v, tiles_v,
                stage0, stage1, posb0, posb1,
                sem_t, sem_s0, sem_s1):
    wid = lax.axis_index("s") * _NUM_CORES + lax.axis_index("c")
    blk0 = wid * _Q + jnp.minimum(wid, _R)
    nblk = jnp.where(wid < _R, _Q + 1, _Q)
    iota = lax.iota(jnp.int32, _L)
    dump = jnp.int32(n)
    nch = n // _CHSZ

    stages = (stage0, stage1)
    posbs = (posb0, posb1)
    sems = (sem_s0, sem_s1)

    for sb in range(2):
        posbs[sb][0, :] = jnp.full((_L,), dump, jnp.int32)
        pltpu.async_copy(stages[sb],
                         out_hbm.at[posbs[sb].at[0]], sems[sb])

    def round_body(carry):
        skip, _prev = carry

        def ch_body(ch, mcount):
            pltpu.sync_copy(idx_hbm.at[pl.ds(ch * _CHSZ, _CHSZ)], idx_v)

            def g_body(g, mc):
                v = idx_v[pl.ds(g * _L, _L)]
                u = lax.shift_right_logical(v, 7) - blk0
                mask = (u >= 0) & (u < nblk)
                mi = jnp.where(mask, 1, 0)
                rank = plsc.cumsum(mi) - 1 + mc
                rel = rank - skip
                m2 = mask & (rel >= 0) & (rel < _CAP)
                soff = jnp.clip(mc - skip, 0, _CAP)
                plsc.store_compressed(sel_i.at[pl.ds(soff, _L)], v, mask=m2)
                pos = iota + (ch * _CHSZ + g * _L)
                plsc.store_compressed(sel_p.at[pl.ds(soff, _L)], pos,
                                      mask=m2)
                return mc + plsc.all_reduce_population_count(mask)[0]

            return lax.fori_loop(0, _CHSZ // _L, g_body, mcount)

        mtotal = lax.fori_loop(0, nch, ch_body, jnp.int32(0))
        this_n = jnp.clip(mtotal - skip, 0, _CAP)
        ngrp_sel = (this_n + _L - 1) // _L

        zeros = jnp.zeros((_L,), jnp.int32)
        for h in range(256 // _L):
            hist_v[pl.ds(h * _L, _L)] = zeros

        def h_body(g, c):
            v = sel_i[pl.ds(g * _L, _L)]
            b = lax.shift_right_logical(v, 7) - blk0
            valid = (iota + g * _L) < this_n
            bc = jnp.where(valid, b, 255)
            cnt, last = plsc.scan_count(bc)
            plsc.addupdate_scatter(hist_v, [bc], cnt, mask=last & valid)
            return c

        lax.fori_loop(0, ngrp_sel, h_body, jnp.int32(0))

        def p_body(h, run):
            v = hist_v[pl.ds(h * _L, _L)]
            cs = plsc.cumsum(v)
            excl = cs - v + run
            offs_v[pl.ds(h * _L, _L)] = excl
            cur_v[pl.ds(h * _L, _L)] = excl
            return run + cs[_L - 1]

        lax.fori_loop(0, 256 // _L, p_body, jnp.int32(0))

        def s_body(g, c):
            v = sel_i[pl.ds(g * _L, _L)]
            p = sel_p[pl.ds(g * _L, _L)]
            b = lax.shift_right_logical(v, 7) - blk0
            valid = (iota + g * _L) < this_n
            bc = jnp.where(valid, b, 255)
            cnt, last = plsc.scan_count(bc)
            base = plsc.load_gather(cur_v, [bc])
            dest = base + cnt - 1
            plsc.store_scatter(srt_i, [dest], v, mask=valid)
            plsc.store_scatter(srt_p, [dest], p, mask=valid)
            plsc.addupdate_scatter(cur_v, [bc], cnt, mask=last)
            return c

        lax.fori_loop(0, ngrp_sel, s_body, jnp.int32(0))

        def extract_sub(s0, end, gstart, stage, posb, sem):
            pltpu.make_async_copy(
                stage, out_hbm.at[posb.at[0]], sem).wait()
            valid = iota < (end - s0)
            v16 = srt_i[pl.ds(s0, _L)]
            p16 = srt_p[pl.ds(s0, _L)]
            lvec = jnp.where(valid, v16 - gstart, 0)
            posb[0, :] = jnp.where(valid, p16, dump)
            for cc in range(_DIM):
                ccv = jnp.full((_L,), cc, jnp.int32)
                vals = plsc.load_gather(tiles_v, [ccv, lvec])
                plsc.store_scatter(stage, [iota, ccv], vals)
            pltpu.async_copy(stage, out_hbm.at[posb.at[0]], sem)

        def grp_body(g, c):
            b0 = g * _GB
            b0c = jnp.minimum(b0, nblk)
            b1c = jnp.minimum(b0 + _GB, nblk)
            gstart = jnp.minimum((blk0 + b0c) * 128, (_NBLK - _GB) * 128)
            gstart = pl.multiple_of(gstart, 128)
            begin = plsc.load_gather(
                offs_v, [jnp.full((_L,), b0c, jnp.int32)])[0]
            end = plsc.load_gather(
                offs_v, [jnp.full((_L,), b1c, jnp.int32)])[0]

            @pl.when(end > begin)
            def _():
                pltpu.sync_copy(table_t.at[:, pl.ds(gstart, _GW)], tiles_v)
                nsub = (end - begin + _L - 1) // _L
                npair = (nsub + 1) // 2

                def pair_body(t, c2):
                    sa = begin + (2 * t) * _L
                    extract_sub(sa, end, gstart, stage0, posb0, sem_s0)
                    sb = sa + _L

                    @pl.when(sb < end)
                    def _():
                        extract_sub(sb, end, gstart, stage1, posb1, sem_s1)

                    return c2

                lax.fori_loop(0, npair, pair_body, jnp.int32(0))

            return c

        lax.fori_loop(0, _NGRP, grp_body, jnp.int32(0))
        return (skip + this_n, this_n)

    def round_cond(carry):
        return carry[1] >= _CAP

    lax.while_loop(round_cond, round_body, (jnp.int32(0), jnp.int32(_CAP)))

    for sb in range(2):
        pltpu.make_async_copy(stages[sb],
                              out_hbm.at[posbs[sb].at[0]], sems[sb]).wait()


@jax.jit
def kernel(x, table):
    batch, seq = x.shape
    vocab, dim = table.shape
    n = batch * seq

    idx = x.reshape(n).astype(jnp.int32)
    table_t = table.T

    n_out = n + 8

    mesh = plsc.VectorSubcoreMesh(
        core_axis_name="c", subcore_axis_name="s",
        num_cores=_NUM_CORES, num_subcores=_NUM_SUBCORES)

    out = pl.kernel(
        functools.partial(_emb_kernel, n),
        out_type=jax.ShapeDtypeStruct((n_out, 128), jnp.float32),
        mesh=mesh,
        scratch_types=[
            pltpu.VMEM((_CHSZ,), jnp.int32),
            pltpu.VMEM((_CAP + _L,), jnp.int32),
            pltpu.VMEM((_CAP + _L,), jnp.int32),
            pltpu.VMEM((_CAP + _L,), jnp.int32),
            pltpu.VMEM((_CAP + _L,), jnp.int32),
            pltpu.VMEM((256,), jnp.int32),
            pltpu.VMEM((256,), jnp.int32),
            pltpu.VMEM((256,), jnp.int32),
            pltpu.VMEM((_DIM, _GW), jnp.float32),
            pltpu.VMEM((_L, 128), jnp.float32),
            pltpu.VMEM((_L, 128), jnp.float32),
            pltpu.VMEM((1, _L), jnp.int32),
            pltpu.VMEM((1, _L), jnp.int32),
            pltpu.SemaphoreType.DMA,
            pltpu.SemaphoreType.DMA,
            pltpu.SemaphoreType.DMA,
        ],
        compiler_params=pltpu.CompilerParams(
            needs_layout_passes=False, disable_bounds_checks=True),
    )(table_t, idx)

    return out[:n, :dim].reshape(batch, seq, dim)
